# baseline (speedup 1.0000x reference)
import functools

import jax
import jax.numpy as jnp
from jax import lax
from jax.experimental import pallas as pl
from jax.experimental.pallas import tpu as pltpu
from jax.experimental.pallas import tpu_sc as plsc

NC = 2
NS = 16
NW = NC * NS
LANES = 16
NBUF = 4
OBUF = 2


@functools.partial(jax.jit, static_argnums=(3, 4, 5, 6))
def _emb_lookup(xT, table, pos, B, S, V, D):
    bpw = B // NW

    mesh = plsc.VectorSubcoreMesh(core_axis_name="c", subcore_axis_name="s")

    @functools.partial(
        pl.kernel,
        out_type=jax.ShapeDtypeStruct((S, D, B), jnp.float32),
        mesh=mesh,
        scratch_types=[
            pltpu.VMEM((S, bpw), jnp.int32),
            pltpu.VMEM((S, D), jnp.float32),
            pltpu.VMEM((NBUF, bpw, D), jnp.float32),
            pltpu.VMEM((OBUF, D, bpw), jnp.float32),
            pltpu.SemaphoreType.DMA,
            pltpu.SemaphoreType.DMA,
        ],
        compiler_params=pltpu.CompilerParams(
            use_tc_tiling_on_sc=False, needs_layout_passes=False),
    )
    def k(xT_hbm, table_hbm, pos_hbm, out_hbm, idx_v, pos_v, rows_v, obuf_v,
          gsem, osem):
        wid = lax.axis_index("s") * NC + lax.axis_index("c")
        b0 = wid * bpw
        pltpu.sync_copy(xT_hbm.at[:, pl.ds(b0, bpw)], idx_v)
        pltpu.sync_copy(pos_hbm, pos_v)

        def gather_parts(s):
            slot = lax.rem(s, NBUF)
            return table_hbm.at[idx_v.at[s]], rows_v.at[slot]

        def store_parts(s):
            oslot = lax.rem(s, OBUF)
            return obuf_v.at[oslot], out_hbm.at[s, :, pl.ds(b0, bpw)]

        for p in range(NBUF - 1):
            src, dst = gather_parts(p)
            pltpu.async_copy(src, dst, gsem)

        row_lanes = lax.iota(jnp.int32, LANES)

        @pl.loop(0, S)
        def _(s):
            slot = lax.rem(s, NBUF)
            oslot = lax.rem(s, OBUF)
            src, dst = gather_parts(s)
            pltpu.make_async_copy(src, dst, gsem).wait()

            @pl.when(s + NBUF - 1 < S)
            def _():
                src2, dst2 = gather_parts(s + NBUF - 1)
                pltpu.async_copy(src2, dst2, gsem)

            @pl.when(s >= OBUF)
            def _():
                src3, dst3 = store_parts(s - OBUF)
                pltpu.make_async_copy(src3, dst3, osem).wait()

            pos_chunks = [pos_v[s, pl.ds(dc * LANES, LANES)]
                          for dc in range(D // LANES)]

            @plsc.parallel_loop(0, bpw, unroll=4)
            def _(b):
                col = jnp.full((LANES,), 0, jnp.int32) + b
                for dc in range(D // LANES):
                    vals = rows_v[slot, b, pl.ds(dc * LANES, LANES)]
                    vals = vals + pos_chunks[dc]
                    rows16 = row_lanes + (dc * LANES)
                    plsc.store_scatter(obuf_v.at[oslot], [rows16, col], vals)

            src4, dst4 = store_parts(s)
            pltpu.async_copy(src4, dst4, osem)

        for t in range(OBUF):
            src5, dst5 = store_parts(S - OBUF + t)
            pltpu.make_async_copy(src5, dst5, osem).wait()

    return k(xT, table, pos)


def kernel(x, table, pos_enc):
    B, S = x.shape
    V, D = table.shape
    xT = jnp.transpose(x.astype(jnp.int32))
    pos = pos_enc[0, :S, :]
    out = _emb_lookup(xT, table, pos, B, S, V, D)
    return jnp.transpose(out, (2, 0, 1))


# speedup vs baseline: 1.0013x; 1.0013x over previous
import functools

import jax
import jax.numpy as jnp
from jax import lax
from jax.experimental import pallas as pl
from jax.experimental.pallas import tpu as pltpu
from jax.experimental.pallas import tpu_sc as plsc

NC = 2
NS = 16
NW = NC * NS
LANES = 16
NBUF = 4
OBUF = 2


@functools.partial(jax.jit, static_argnums=(3, 4, 5, 6))
def _emb_lookup(xT, table, pos, B, S, V, D):
    bpw = B // NW

    mesh = plsc.VectorSubcoreMesh(core_axis_name="c", subcore_axis_name="s")

    @functools.partial(
        pl.kernel,
        out_type=jax.ShapeDtypeStruct((S, D, B), jnp.float32),
        mesh=mesh,
        scratch_types=[
            pltpu.VMEM((S, bpw), jnp.int32),
            pltpu.VMEM((S, D), jnp.float32),
            pltpu.VMEM((NBUF, bpw, D), jnp.float32),
            pltpu.VMEM((OBUF, D, bpw), jnp.float32),
            pltpu.SemaphoreType.DMA,
            pltpu.SemaphoreType.DMA,
        ],
        compiler_params=pltpu.CompilerParams(
            use_tc_tiling_on_sc=False, needs_layout_passes=False,
            disable_bounds_checks=True),
    )
    def k(xT_hbm, table_hbm, pos_hbm, out_hbm, idx_v, pos_v, rows_v, obuf_v,
          gsem, osem):
        wid = lax.axis_index("s") * NC + lax.axis_index("c")
        b0 = wid * bpw
        pltpu.sync_copy(xT_hbm.at[:, pl.ds(b0, bpw)], idx_v)
        pltpu.sync_copy(pos_hbm, pos_v)

        def gather_parts(s):
            slot = lax.rem(s, NBUF)
            return table_hbm.at[idx_v.at[s]], rows_v.at[slot]

        def store_parts(s):
            oslot = lax.rem(s, OBUF)
            return obuf_v.at[oslot], out_hbm.at[s, :, pl.ds(b0, bpw)]

        for p in range(NBUF - 1):
            src, dst = gather_parts(p)
            pltpu.async_copy(src, dst, gsem)

        lane_rows = lax.iota(jnp.int32, LANES)
        zeros16 = lane_rows * 0

        @pl.loop(0, S)
        def _(s):
            slot = lax.rem(s, NBUF)
            oslot = lax.rem(s, OBUF)
            src, dst = gather_parts(s)
            pltpu.make_async_copy(src, dst, gsem).wait()

            @pl.when(s + NBUF - 1 < S)
            def _():
                src2, dst2 = gather_parts(s + NBUF - 1)
                pltpu.async_copy(src2, dst2, gsem)

            @pl.when(s >= OBUF)
            def _():
                src3, dst3 = store_parts(s - OBUF)
                pltpu.make_async_copy(src3, dst3, osem).wait()

            pos_chunks = [pos_v[s, pl.ds(dc * LANES, LANES)]
                          for dc in range(D // LANES)]

            obuf2 = obuf_v.at[oslot]
            row_idx = [lane_rows + (dc * LANES) for dc in range(D // LANES)]

            @plsc.parallel_loop(0, bpw, unroll=4)
            def _(b):
                col = zeros16 + b
                for dc in range(D // LANES):
                    vals = rows_v[slot, b, pl.ds(dc * LANES, LANES)]
                    vals = vals + pos_chunks[dc]
                    plsc.store_scatter(obuf2, [row_idx[dc], col], vals)

            src4, dst4 = store_parts(s)
            pltpu.async_copy(src4, dst4, osem)

        for t in range(OBUF):
            src5, dst5 = store_parts(S - OBUF + t)
            pltpu.make_async_copy(src5, dst5, osem).wait()

    return k(xT, table, pos)


def kernel(x, table, pos_enc):
    B, S = x.shape
    V, D = table.shape
    xT = jnp.transpose(x.astype(jnp.int32))
    pos = pos_enc[0, :S, :]
    out = _emb_lookup(xT, table, pos, B, S, V, D)
    return jnp.transpose(out, (2, 0, 1))


# speedup vs baseline: 1.0070x; 1.0057x over previous
import functools

import jax
import jax.numpy as jnp
from jax import lax
from jax.experimental import pallas as pl
from jax.experimental.pallas import tpu as pltpu
from jax.experimental.pallas import tpu_sc as plsc

NC = 2
NS = 16
NW = NC * NS
LANES = 16
NBUF = 4
OBUF = 2


@functools.partial(jax.jit, static_argnums=(3, 4, 5, 6))
def _emb_lookup(xT, table, pos, B, S, V, D):
    bpw = B // NW

    mesh = plsc.VectorSubcoreMesh(core_axis_name="c", subcore_axis_name="s")

    @functools.partial(
        pl.kernel,
        out_type=jax.ShapeDtypeStruct((S, D, B), jnp.float32),
        mesh=mesh,
        scratch_types=[
            pltpu.VMEM((S, bpw), jnp.int32),
            pltpu.VMEM((S, D), jnp.float32),
            pltpu.VMEM((NBUF, bpw, D), jnp.float32),
            pltpu.VMEM((OBUF, D, bpw), jnp.float32),
            pltpu.SemaphoreType.DMA,
            pltpu.SemaphoreType.DMA,
        ],
        compiler_params=pltpu.CompilerParams(
            use_tc_tiling_on_sc=False, needs_layout_passes=False,
            disable_bounds_checks=True),
    )
    def k(xT_hbm, table_hbm, pos_hbm, out_hbm, idx_v, pos_v, rows_v, obuf_v,
          gsem, osem):
        wid = lax.axis_index("s") * NC + lax.axis_index("c")
        b0 = wid * bpw
        pltpu.sync_copy(xT_hbm.at[:, pl.ds(b0, bpw)], idx_v)
        pltpu.sync_copy(pos_hbm, pos_v)

        def gather_parts(s):
            slot = lax.rem(s, NBUF)
            return table_hbm.at[idx_v.at[s]], rows_v.at[slot]

        def store_parts(s):
            oslot = lax.rem(s, OBUF)
            return obuf_v.at[oslot], out_hbm.at[s, :, pl.ds(b0, bpw)]

        for p in range(NBUF - 1):
            src, dst = gather_parts(p)
            pltpu.async_copy(src, dst, gsem)

        lane_rows = lax.iota(jnp.int32, LANES)
        zeros16 = lane_rows * 0
        b_vecs = [lane_rows + bc * LANES for bc in range(bpw // LANES)]

        @pl.loop(0, S)
        def _(s):
            slot = lax.rem(s, NBUF)
            oslot = lax.rem(s, OBUF)
            src, dst = gather_parts(s)
            pltpu.make_async_copy(src, dst, gsem).wait()

            @pl.when(s + NBUF - 1 < S)
            def _():
                src2, dst2 = gather_parts(s + NBUF - 1)
                pltpu.async_copy(src2, dst2, gsem)

            @pl.when(s >= OBUF)
            def _():
                src3, dst3 = store_parts(s - OBUF)
                pltpu.make_async_copy(src3, dst3, osem).wait()

            rows2 = rows_v.at[slot]
            obuf2 = obuf_v.at[oslot]
            s_splat = zeros16 + s

            @plsc.parallel_loop(0, D, unroll=4)
            def _(d):
                d_splat = zeros16 + d
                posb = plsc.load_gather(pos_v, [s_splat, d_splat])
                for bc in range(bpw // LANES):
                    vals = plsc.load_gather(rows2, [b_vecs[bc], d_splat])
                    obuf2[d, pl.ds(bc * LANES, LANES)] = vals + posb

            src4, dst4 = store_parts(s)
            pltpu.async_copy(src4, dst4, osem)

        for t in range(OBUF):
            src5, dst5 = store_parts(S - OBUF + t)
            pltpu.make_async_copy(src5, dst5, osem).wait()

    return k(xT, table, pos)


def kernel(x, table, pos_enc):
    B, S = x.shape
    V, D = table.shape
    xT = jnp.transpose(x.astype(jnp.int32))
    pos = pos_enc[0, :S, :]
    out = _emb_lookup(xT, table, pos, B, S, V, D)
    return jnp.transpose(out, (2, 0, 1))


# speedup vs baseline: 1.1866x; 1.1783x over previous
import functools

import jax
import jax.numpy as jnp
from jax import lax
from jax.experimental import pallas as pl
from jax.experimental.pallas import tpu as pltpu
from jax.experimental.pallas import tpu_sc as plsc

NC = 2
NS = 16
NW = NC * NS
LANES = 16
NBUF = 4


@functools.partial(jax.jit, static_argnums=(3, 4, 5, 6))
def _emb_lookup(x2d, table, pos, B, S, V, D):
    HS = S // 2
    seqs_per_w = B // NW

    mesh = plsc.VectorSubcoreMesh(core_axis_name="c", subcore_axis_name="s")

    @functools.partial(
        pl.kernel,
        out_type=jax.ShapeDtypeStruct((B, S, D), jnp.float32),
        mesh=mesh,
        scratch_types=[
            pltpu.VMEM((2 * seqs_per_w, HS), jnp.int32),
            pltpu.VMEM((S, D), jnp.float32),
            pltpu.VMEM((NBUF, S, D), jnp.float32),
            pltpu.SemaphoreType.DMA,
            pltpu.SemaphoreType.DMA,
        ],
        compiler_params=pltpu.CompilerParams(use_tc_tiling_on_sc=False),
    )
    def k(x_hbm, table_hbm, pos_hbm, out_hbm, idx_v, pos_v, rows_v, gsem, osem):
        wid = lax.axis_index("s") * NC + lax.axis_index("c")
        base_seq = wid * seqs_per_w
        pltpu.sync_copy(x_hbm.at[pl.ds(wid * 2 * seqs_per_w, 2 * seqs_per_w)], idx_v)
        pltpu.sync_copy(pos_hbm, pos_v)

        def gather_parts(i):
            slot = lax.rem(i, NBUF)
            return (
                (table_hbm.at[idx_v.at[2 * i]], rows_v.at[slot, pl.ds(0, HS)]),
                (table_hbm.at[idx_v.at[2 * i + 1]], rows_v.at[slot, pl.ds(HS, HS)]),
            )

        def store_parts(i):
            slot = lax.rem(i, NBUF)
            return rows_v.at[slot], out_hbm.at[base_seq + i]

        def start_gather(i):
            for src, dst in gather_parts(i):
                pltpu.async_copy(src, dst, gsem)

        for p in range(NBUF - 1):
            start_gather(p)

        @pl.loop(0, seqs_per_w)
        def _(i):
            slot = lax.rem(i, NBUF)
            for src, dst in gather_parts(i):
                pltpu.make_async_copy(src, dst, gsem).wait()

            @pl.when(i >= 1)
            def _():
                src, dst = store_parts(i - 1)
                pltpu.make_async_copy(src, dst, osem).wait()

            @pl.when(i + NBUF - 1 < seqs_per_w)
            def _():
                start_gather(i + NBUF - 1)

            @plsc.parallel_loop(0, S, unroll=4)
            def _(r):
                for d in range(0, D, LANES):
                    rows_v[slot, r, pl.ds(d, LANES)] = (
                        rows_v[slot, r, pl.ds(d, LANES)] + pos_v[r, pl.ds(d, LANES)])

            src, dst = store_parts(i)
            pltpu.async_copy(src, dst, osem)

        src, dst = store_parts(seqs_per_w - 1)
        pltpu.make_async_copy(src, dst, osem).wait()

    return k(x2d, table, pos)


def kernel(x, table, pos_enc):
    B, S = x.shape
    V, D = table.shape
    x2d = x.astype(jnp.int32).reshape(B * S // (S // 2), S // 2)
    pos = pos_enc[0, :S, :]
    return _emb_lookup(x2d, table, pos, B, S, V, D)
